# Initial kernel scaffold; baseline (speedup 1.0000x reference)
import jax
import jax.numpy as jnp
from jax.experimental import pallas as pl


def kernel(x, edge_index, W1, b1, W2, b2):
    raise NotImplementedError("write your pallas kernel here")


# speedup vs baseline: 16.4578x; 16.4578x over previous
import functools

import jax
import jax.numpy as jnp
from jax import lax
from jax.experimental import pallas as pl
from jax.experimental.pallas import tpu as pltpu
from jax.experimental.pallas import tpu_sc as plsc

N = 10000
E = 320000
D = 128
CHUNK = 128
NCHUNKS = E // CHUNK
NC = 2
NS = 16
NW = NC * NS
FULL = NCHUNKS // NW
REM = NCHUNKS - FULL * NW

@functools.cache
def _make_deg_kernel():
    mesh = plsc.VectorSubcoreMesh(core_axis_name="c", subcore_axis_name="s")
    return functools.partial(
        pl.kernel,
        mesh=mesh,
        out_type=jax.ShapeDtypeStruct((2 * N,), jnp.float32),
        scratch_types=[
            pltpu.VMEM((CHUNK,), jnp.int32),
            pltpu.VMEM((CHUNK,), jnp.float32),
            pltpu.VMEM((N,), jnp.float32),
            pltpu.VMEM_SHARED((N,), jnp.float32),
        ],
    )(_deg_body)


def _deg_body(dst_hbm, zeros_hbm, out_hbm, idx_v, ones_v, bounce_v, acc):
    c = lax.axis_index("c")
    s = lax.axis_index("s")
    wid = s * NC + c

    for k in range(CHUNK // 16):
        ones_v[pl.ds(k * 16, 16)] = jnp.ones((16,), jnp.float32)

    @pl.when(s == 0)
    def _():
        pltpu.sync_copy(zeros_hbm, bounce_v)
        pltpu.sync_copy(bounce_v, acc)

    plsc.subcore_barrier()

    def step(j, carry):
        chunk = wid + j * NW
        pltpu.sync_copy(dst_hbm.at[chunk], idx_v)
        pltpu.sync_copy(ones_v, acc.at[idx_v], add=True)
        return carry

    lax.fori_loop(0, FULL, step, 0)

    @pl.when(wid < REM)
    def _():
        step(FULL, 0)

    plsc.subcore_barrier()

    @pl.when(s == 0)
    def _():
        pltpu.sync_copy(acc, bounce_v)
        pltpu.sync_copy(bounce_v, out_hbm.at[pl.ds(c * N, N)])


@functools.cache
def _make_gs_kernel():
    mesh = plsc.VectorSubcoreMesh(core_axis_name="c", subcore_axis_name="s")
    return functools.partial(
        pl.kernel,
        mesh=mesh,
        out_type=jax.ShapeDtypeStruct((2 * N, D), jnp.float32),
        scratch_types=[
            pltpu.VMEM((CHUNK,), jnp.int32),
            pltpu.VMEM((CHUNK,), jnp.int32),
            pltpu.VMEM((CHUNK, D), jnp.float32),
            pltpu.VMEM_SHARED((N, D), jnp.float32),
            pltpu.SemaphoreType.DMA,
        ],
    )(_gs_body)


def _gs_body(src_hbm, dst_hbm, table_hbm, zeros_hbm, out_hbm,
             idx_s, idx_d, rows, acc, sem):
    c = lax.axis_index("c")
    s = lax.axis_index("s")
    wid = s * NC + c

    @pl.when(s == 0)
    def _():
        pltpu.sync_copy(zeros_hbm, acc)

    plsc.subcore_barrier()

    def step(j, carry):
        chunk = wid + j * NW
        pltpu.sync_copy(src_hbm.at[chunk], idx_s)
        pltpu.sync_copy(dst_hbm.at[chunk], idx_d)
        pltpu.async_copy(table_hbm.at[idx_s], rows, sem).wait()
        pltpu.sync_copy(rows, acc.at[idx_d], add=True)
        return carry

    lax.fori_loop(0, FULL, step, 0)

    @pl.when(wid < REM)
    def _():
        step(FULL, 0)

    plsc.subcore_barrier()

    @pl.when(s < NS - 1)
    def _():
        base = s * 640
        pltpu.sync_copy(acc.at[pl.ds(base, 640)],
                        out_hbm.at[pl.ds(c * N + base, 640)])

    @pl.when(s == NS - 1)
    def _():
        pltpu.sync_copy(acc.at[pl.ds(9600, 400)],
                        out_hbm.at[pl.ds(c * N + 9600, 400)])


_BLK = 1000
_GRID = N // _BLK


def _tc_a_body(x_ref, w_ref, d0_ref, d1_ref, g_ref, dinv_ref):
    deg = d0_ref[...] + d1_ref[...] + 1.0
    dinv = lax.rsqrt(deg)
    g_ref[...] = jnp.dot(x_ref[...], w_ref[...],
                         preferred_element_type=jnp.float32) * dinv
    dinv_ref[...] = dinv


def _tc_a(x, W1, degp):
    return pl.pallas_call(
        _tc_a_body,
        grid=(_GRID,),
        in_specs=[
            pl.BlockSpec((_BLK, D), lambda i: (i, 0)),
            pl.BlockSpec((D, D), lambda i: (0, 0)),
            pl.BlockSpec((_BLK, 1), lambda i: (i, 0)),
            pl.BlockSpec((_BLK, 1), lambda i: (i + _GRID, 0)),
        ],
        out_specs=[
            pl.BlockSpec((_BLK, D), lambda i: (i, 0)),
            pl.BlockSpec((_BLK, 1), lambda i: (i, 0)),
        ],
        out_shape=[
            jax.ShapeDtypeStruct((N, D), jnp.float32),
            jax.ShapeDtypeStruct((N, 1), jnp.float32),
        ],
    )(x, W1, degp, degp)


def _tc_b_body(s0_ref, s1_ref, g1_ref, dinv_ref, b1_ref, w2_ref, g2_ref):
    dinv = dinv_ref[...]
    h = dinv * (s0_ref[...] + s1_ref[...] + g1_ref[...]) + b1_ref[...]
    h = jnp.maximum(h, 0.0)
    g2_ref[...] = jnp.dot(h, w2_ref[...],
                          preferred_element_type=jnp.float32) * dinv


def _tc_b(s1, g1, dinv, b1r, W2):
    return pl.pallas_call(
        _tc_b_body,
        grid=(_GRID,),
        in_specs=[
            pl.BlockSpec((_BLK, D), lambda i: (i, 0)),
            pl.BlockSpec((_BLK, D), lambda i: (i + _GRID, 0)),
            pl.BlockSpec((_BLK, D), lambda i: (i, 0)),
            pl.BlockSpec((_BLK, 1), lambda i: (i, 0)),
            pl.BlockSpec((1, D), lambda i: (0, 0)),
            pl.BlockSpec((D, D), lambda i: (0, 0)),
        ],
        out_specs=pl.BlockSpec((_BLK, D), lambda i: (i, 0)),
        out_shape=jax.ShapeDtypeStruct((N, D), jnp.float32),
    )(s1, s1, g1, dinv, b1r, W2)


def _tc_c_body(s0_ref, s1_ref, g2_ref, dinv_ref, b2_ref, out_ref):
    out_ref[...] = (dinv_ref[...] * (s0_ref[...] + s1_ref[...] + g2_ref[...])
                    + b2_ref[...])


def _tc_c(s2, g2, dinv, b2r):
    return pl.pallas_call(
        _tc_c_body,
        grid=(_GRID,),
        in_specs=[
            pl.BlockSpec((_BLK, D), lambda i: (i, 0)),
            pl.BlockSpec((_BLK, D), lambda i: (i + _GRID, 0)),
            pl.BlockSpec((_BLK, D), lambda i: (i, 0)),
            pl.BlockSpec((_BLK, 1), lambda i: (i, 0)),
            pl.BlockSpec((1, D), lambda i: (0, 0)),
        ],
        out_specs=pl.BlockSpec((_BLK, D), lambda i: (i, 0)),
        out_shape=jax.ShapeDtypeStruct((N, D), jnp.float32),
    )(s2, s2, g2, dinv, b2r)


def kernel(x, edge_index, W1, b1, W2, b2):
    src = edge_index[0].astype(jnp.int32).reshape(NCHUNKS, CHUNK)
    dst = edge_index[1].astype(jnp.int32).reshape(NCHUNKS, CHUNK)
    zeros_vec = jnp.zeros((N,), jnp.float32)
    zeros_tab = jnp.zeros((N, D), jnp.float32)
    b1r = b1.reshape(1, D)
    b2r = b2.reshape(1, D)

    deg_kernel = _make_deg_kernel()
    gs_kernel = _make_gs_kernel()
    degp = deg_kernel(dst, zeros_vec).reshape(2 * N, 1)
    g1, dinv = _tc_a(x, W1, degp)
    s1 = gs_kernel(src, dst, g1, zeros_tab)
    g2 = _tc_b(s1, g1, dinv, b1r, W2)
    s2 = gs_kernel(src, dst, g2, zeros_tab)
    return _tc_c(s2, g2, dinv, b2r)


# speedup vs baseline: 27.8473x; 1.6920x over previous
import functools

import jax
import jax.numpy as jnp
from jax import lax
from jax.experimental import pallas as pl
from jax.experimental.pallas import tpu as pltpu
from jax.experimental.pallas import tpu_sc as plsc

N = 10000
E = 320000
D = 128
CHUNK = 128
NC = 2
NS = 16
NW = NC * NS
EPB = 80
NCHUNKS_P = NW * EPB
EPAD = NCHUNKS_P * CHUNK - E
NREAL_CHUNKS = E // CHUNK
PADROWS = 64
NTAB = N + PADROWS
QCH = 40


@functools.cache
def _make_deg_kernel():
    mesh = plsc.VectorSubcoreMesh(core_axis_name="c", subcore_axis_name="s")
    return functools.partial(
        pl.kernel,
        mesh=mesh,
        out_type=jax.ShapeDtypeStruct((2 * N,), jnp.float32),
        scratch_types=[
            pltpu.VMEM((EPB, CHUNK), jnp.int32),
            pltpu.VMEM((CHUNK,), jnp.float32),
            pltpu.VMEM((N,), jnp.float32),
            pltpu.VMEM_SHARED((N,), jnp.float32),
            pltpu.SemaphoreType.DMA,
        ],
    )(_deg_body)


def _deg_body(dst_hbm, zeros_hbm, out_hbm, idx_d, ones_v, bounce_v, acc, sem):
    c = lax.axis_index("c")
    s = lax.axis_index("s")
    wid = s * NC + c

    for k in range(CHUNK // 16):
        ones_v[pl.ds(k * 16, 16)] = jnp.ones((16,), jnp.float32)

    pltpu.sync_copy(dst_hbm.at[pl.ds(wid * EPB, EPB)], idx_d)

    @pl.when(s == 0)
    def _():
        pltpu.sync_copy(zeros_hbm, bounce_v)
        pltpu.sync_copy(bounce_v, acc)

    plsc.subcore_barrier()

    nreal = jnp.clip(NREAL_CHUNKS - wid * EPB, 0, EPB)

    def fire(j, carry):
        pltpu.sync_copy(ones_v, acc.at[idx_d.at[j]], add=True)
        return carry

    lax.fori_loop(0, nreal, fire, 0)

    plsc.subcore_barrier()

    @pl.when(s == 0)
    def _():
        pltpu.sync_copy(acc, bounce_v)
        pltpu.sync_copy(bounce_v, out_hbm.at[pl.ds(c * N, N)])


@functools.cache
def _make_gs_kernel():
    mesh = plsc.VectorSubcoreMesh(core_axis_name="c", subcore_axis_name="s")
    return functools.partial(
        pl.kernel,
        mesh=mesh,
        out_type=jax.ShapeDtypeStruct((2 * N, D), jnp.float32),
        scratch_types=[
            pltpu.VMEM((QCH, CHUNK), jnp.int32),
            pltpu.VMEM((QCH, CHUNK), jnp.int32),
            pltpu.VMEM((CHUNK, D), jnp.float32),
            pltpu.VMEM((CHUNK, D), jnp.float32),
            pltpu.VMEM_SHARED((N, D), jnp.float32),
            pltpu.SemaphoreType.DMA,
        ],
    )(_gs_body)


def _gs_body(src_hbm, dst_hbm, table_hbm, zeros_hbm, out_hbm,
             idx_s, idx_d, rows0, rows1, acc, gsem):
    c = lax.axis_index("c")
    s = lax.axis_index("s")
    wid = s * NC + c

    @pl.when(s < NS - 1)
    def _():
        pltpu.sync_copy(zeros_hbm.at[pl.ds(s * 640, 640)],
                        acc.at[pl.ds(s * 640, 640)])

    @pl.when(s == NS - 1)
    def _():
        pltpu.sync_copy(zeros_hbm.at[pl.ds(9600, N - 9600)],
                        acc.at[pl.ds(9600, N - 9600)])

    plsc.subcore_barrier()

    for q in range(EPB // QCH):
        qbase = wid * EPB + q * QCH
        pltpu.sync_copy(src_hbm.at[pl.ds(qbase, QCH)], idx_s)
        pltpu.sync_copy(dst_hbm.at[pl.ds(qbase, QCH)], idx_d)

        pltpu.async_copy(table_hbm.at[idx_s.at[0]], rows0, gsem)

        def outer(i, carry):
            j = 2 * i
            pltpu.make_async_copy(table_hbm.at[idx_s.at[j]], rows0, gsem).wait()
            pltpu.async_copy(table_hbm.at[idx_s.at[j + 1]], rows1, gsem)
            pltpu.sync_copy(rows0, acc.at[idx_d.at[j]], add=True)

            pltpu.make_async_copy(table_hbm.at[idx_s.at[j + 1]], rows1,
                                  gsem).wait()

            @pl.when(j + 2 < QCH)
            def _():
                pltpu.async_copy(table_hbm.at[idx_s.at[j + 2]], rows0, gsem)

            pltpu.sync_copy(rows1, acc.at[idx_d.at[j + 1]], add=True)
            return carry

        lax.fori_loop(0, QCH // 2, outer, 0)

    plsc.subcore_barrier()

    @pl.when(s < NS - 1)
    def _():
        base = s * 640
        pltpu.sync_copy(acc.at[pl.ds(base, 640)],
                        out_hbm.at[pl.ds(c * N + base, 640)])

    @pl.when(s == NS - 1)
    def _():
        pltpu.sync_copy(acc.at[pl.ds(9600, 400)],
                        out_hbm.at[pl.ds(c * N + 9600, 400)])


_BLK = 1000
_GRID = N // _BLK


def _tc_a_body(x_ref, w_ref, d0_ref, d1_ref, g_ref, dinv_ref):
    deg = d0_ref[...] + d1_ref[...] + 1.0
    dinv = lax.rsqrt(deg)
    g_ref[...] = jnp.dot(x_ref[...], w_ref[...],
                         preferred_element_type=jnp.float32) * dinv
    dinv_ref[...] = dinv


def _tc_a(x, W1, degp):
    return pl.pallas_call(
        _tc_a_body,
        grid=(_GRID,),
        in_specs=[
            pl.BlockSpec((_BLK, D), lambda i: (i, 0)),
            pl.BlockSpec((D, D), lambda i: (0, 0)),
            pl.BlockSpec((_BLK, 1), lambda i: (i, 0)),
            pl.BlockSpec((_BLK, 1), lambda i: (i + _GRID, 0)),
        ],
        out_specs=[
            pl.BlockSpec((_BLK, D), lambda i: (i, 0)),
            pl.BlockSpec((_BLK, 1), lambda i: (i, 0)),
        ],
        out_shape=[
            jax.ShapeDtypeStruct((N, D), jnp.float32),
            jax.ShapeDtypeStruct((N, 1), jnp.float32),
        ],
    )(x, W1, degp, degp)


def _tc_b_body(s0_ref, s1_ref, g1_ref, dinv_ref, b1_ref, w2_ref, g2_ref):
    dinv = dinv_ref[...]
    h = dinv * (s0_ref[...] + s1_ref[...] + g1_ref[...]) + b1_ref[...]
    h = jnp.maximum(h, 0.0)
    g2_ref[...] = jnp.dot(h, w2_ref[...],
                          preferred_element_type=jnp.float32) * dinv


def _tc_b(s1, g1, dinv, b1r, W2):
    return pl.pallas_call(
        _tc_b_body,
        grid=(_GRID,),
        in_specs=[
            pl.BlockSpec((_BLK, D), lambda i: (i, 0)),
            pl.BlockSpec((_BLK, D), lambda i: (i + _GRID, 0)),
            pl.BlockSpec((_BLK, D), lambda i: (i, 0)),
            pl.BlockSpec((_BLK, 1), lambda i: (i, 0)),
            pl.BlockSpec((1, D), lambda i: (0, 0)),
            pl.BlockSpec((D, D), lambda i: (0, 0)),
        ],
        out_specs=pl.BlockSpec((_BLK, D), lambda i: (i, 0)),
        out_shape=jax.ShapeDtypeStruct((N, D), jnp.float32),
    )(s1, s1, g1, dinv, b1r, W2)


def _tc_c_body(s0_ref, s1_ref, g2_ref, dinv_ref, b2_ref, out_ref):
    out_ref[...] = (dinv_ref[...] * (s0_ref[...] + s1_ref[...] + g2_ref[...])
                    + b2_ref[...])


def _tc_c(s2, g2, dinv, b2r):
    return pl.pallas_call(
        _tc_c_body,
        grid=(_GRID,),
        in_specs=[
            pl.BlockSpec((_BLK, D), lambda i: (i, 0)),
            pl.BlockSpec((_BLK, D), lambda i: (i + _GRID, 0)),
            pl.BlockSpec((_BLK, D), lambda i: (i, 0)),
            pl.BlockSpec((_BLK, 1), lambda i: (i, 0)),
            pl.BlockSpec((1, D), lambda i: (0, 0)),
        ],
        out_specs=pl.BlockSpec((_BLK, D), lambda i: (i, 0)),
        out_shape=jax.ShapeDtypeStruct((N, D), jnp.float32),
    )(s2, s2, g2, dinv, b2r)


def kernel(x, edge_index, W1, b1, W2, b2):
    pad_i = jnp.arange(EPAD, dtype=jnp.int32)
    src = jnp.concatenate(
        [edge_index[0].astype(jnp.int32), N + pad_i % PADROWS]
    ).reshape(NCHUNKS_P, CHUNK)
    dst = jnp.concatenate(
        [edge_index[1].astype(jnp.int32), pad_i % N]
    ).reshape(NCHUNKS_P, CHUNK)
    zeros_vec = jnp.zeros((N,), jnp.float32)
    zeros_tab = jnp.zeros((N, D), jnp.float32)
    tab_pad = jnp.zeros((PADROWS, D), jnp.float32)
    b1r = b1.reshape(1, D)
    b2r = b2.reshape(1, D)

    deg_kernel = _make_deg_kernel()
    gs_kernel = _make_gs_kernel()
    degp = deg_kernel(dst, zeros_vec).reshape(2 * N, 1)
    g1, dinv = _tc_a(x, W1, degp)
    s1 = gs_kernel(src, dst, jnp.concatenate([g1, tab_pad]), zeros_tab)
    g2 = _tc_b(s1, g1, dinv, b1r, W2)
    s2 = gs_kernel(src, dst, jnp.concatenate([g2, tab_pad]), zeros_tab)
    return _tc_c(s2, g2, dinv, b2r)


# speedup vs baseline: 27.8508x; 1.0001x over previous
import functools

import jax
import jax.numpy as jnp
from jax import lax
from jax.experimental import pallas as pl
from jax.experimental.pallas import tpu as pltpu
from jax.experimental.pallas import tpu_sc as plsc

N = 10000
E = 320000
D = 128
CHUNK = 128
NC = 2
NS = 16
NW = NC * NS
EPB = 80
NCHUNKS_P = NW * EPB
EPAD = NCHUNKS_P * CHUNK - E
NREAL_CHUNKS = E // CHUNK
PADROWS = 64
NTAB = N + PADROWS
QCH = 40


@functools.cache
def _make_deg_kernel():
    mesh = plsc.VectorSubcoreMesh(core_axis_name="c", subcore_axis_name="s")
    return functools.partial(
        pl.kernel,
        mesh=mesh,
        out_type=jax.ShapeDtypeStruct((2 * N,), jnp.float32),
        scratch_types=[
            pltpu.VMEM((EPB, CHUNK), jnp.int32),
            pltpu.VMEM((CHUNK,), jnp.float32),
            pltpu.VMEM((N,), jnp.float32),
            pltpu.VMEM_SHARED((N,), jnp.float32),
            pltpu.SemaphoreType.DMA,
        ],
    )(_deg_body)


def _deg_body(dst_hbm, zeros_hbm, out_hbm, idx_d, ones_v, bounce_v, acc, sem):
    c = lax.axis_index("c")
    s = lax.axis_index("s")
    wid = s * NC + c

    for k in range(CHUNK // 16):
        ones_v[pl.ds(k * 16, 16)] = jnp.ones((16,), jnp.float32)

    pltpu.sync_copy(dst_hbm.at[pl.ds(wid * EPB, EPB)], idx_d)

    @pl.when(s == 0)
    def _():
        pltpu.sync_copy(zeros_hbm, bounce_v)
        pltpu.sync_copy(bounce_v, acc)

    plsc.subcore_barrier()

    nreal = jnp.clip(NREAL_CHUNKS - wid * EPB, 0, EPB)

    def fire(j, carry):
        pltpu.sync_copy(ones_v, acc.at[idx_d.at[j]], add=True)
        return carry

    lax.fori_loop(0, nreal, fire, 0)

    plsc.subcore_barrier()

    @pl.when(s == 0)
    def _():
        pltpu.sync_copy(acc, bounce_v)
        pltpu.sync_copy(bounce_v, out_hbm.at[pl.ds(c * N, N)])


@functools.cache
def _make_gs_kernel():
    mesh = plsc.VectorSubcoreMesh(core_axis_name="c", subcore_axis_name="s")
    return functools.partial(
        pl.kernel,
        mesh=mesh,
        out_type=jax.ShapeDtypeStruct((2 * N, D), jnp.float32),
        scratch_types=[
            pltpu.VMEM((QCH, CHUNK), jnp.int32),
            pltpu.VMEM((QCH, CHUNK), jnp.int32),
            pltpu.VMEM((CHUNK, D), jnp.float32),
            pltpu.VMEM((CHUNK, D), jnp.float32),
            pltpu.VMEM_SHARED((N, D), jnp.float32),
            pltpu.SemaphoreType.DMA,
        ],
    )(_gs_body)


def _gs_body(src_hbm, dst_hbm, table_hbm, zeros_hbm, out_hbm,
             idx_s, idx_d, rows0, rows1, acc, gsem):
    c = lax.axis_index("c")
    s = lax.axis_index("s")
    wid = s * NC + c

    @pl.when(s < NS - 1)
    def _():
        pltpu.sync_copy(zeros_hbm.at[pl.ds(s * 640, 640)],
                        acc.at[pl.ds(s * 640, 640)])

    @pl.when(s == NS - 1)
    def _():
        pltpu.sync_copy(zeros_hbm.at[pl.ds(9600, N - 9600)],
                        acc.at[pl.ds(9600, N - 9600)])

    plsc.subcore_barrier()

    for q in range(EPB // QCH):
        qbase = wid * EPB + q * QCH
        pltpu.sync_copy(src_hbm.at[pl.ds(qbase, QCH)], idx_s)
        pltpu.sync_copy(dst_hbm.at[pl.ds(qbase, QCH)], idx_d)

        pltpu.async_copy(table_hbm.at[idx_s.at[0]], rows0, gsem)

        def outer(i, carry):
            j = 2 * i
            pltpu.make_async_copy(table_hbm.at[idx_s.at[j]], rows0, gsem).wait()
            pltpu.async_copy(table_hbm.at[idx_s.at[j + 1]], rows1, gsem)
            pltpu.sync_copy(rows0, acc.at[idx_d.at[j]], add=True)

            pltpu.make_async_copy(table_hbm.at[idx_s.at[j + 1]], rows1,
                                  gsem).wait()

            @pl.when(j + 2 < QCH)
            def _():
                pltpu.async_copy(table_hbm.at[idx_s.at[j + 2]], rows0, gsem)

            pltpu.sync_copy(rows1, acc.at[idx_d.at[j + 1]], add=True)
            return carry

        lax.fori_loop(0, QCH // 2, outer, 0)

    plsc.subcore_barrier()

    @pl.when(s < NS - 1)
    def _():
        base = s * 640
        pltpu.sync_copy(acc.at[pl.ds(base, 640)],
                        out_hbm.at[pl.ds(c * N + base, 640)])

    @pl.when(s == NS - 1)
    def _():
        pltpu.sync_copy(acc.at[pl.ds(9600, 400)],
                        out_hbm.at[pl.ds(c * N + 9600, 400)])


_BLK = 1000
_GRID = N // _BLK


def _tc_mm_body(x_ref, w_ref, o_ref):
    o_ref[...] = jnp.dot(x_ref[...], w_ref[...],
                         preferred_element_type=jnp.float32)


def _tc_mm(x, W1):
    return pl.pallas_call(
        _tc_mm_body,
        grid=(_GRID,),
        in_specs=[
            pl.BlockSpec((_BLK, D), lambda i: (i, 0)),
            pl.BlockSpec((D, D), lambda i: (0, 0)),
        ],
        out_specs=pl.BlockSpec((_BLK, D), lambda i: (i, 0)),
        out_shape=jax.ShapeDtypeStruct((N, D), jnp.float32),
    )(x, W1)


def _tc_a_body(hw_ref, d0_ref, d1_ref, g_ref, dinv_ref):
    deg = d0_ref[...] + d1_ref[...] + 1.0
    dinv = lax.rsqrt(deg)
    g_ref[...] = hw_ref[...] * dinv
    dinv_ref[...] = dinv


def _tc_a(hw, degp):
    return pl.pallas_call(
        _tc_a_body,
        grid=(_GRID,),
        in_specs=[
            pl.BlockSpec((_BLK, D), lambda i: (i, 0)),
            pl.BlockSpec((_BLK, 1), lambda i: (i, 0)),
            pl.BlockSpec((_BLK, 1), lambda i: (i + _GRID, 0)),
        ],
        out_specs=[
            pl.BlockSpec((_BLK, D), lambda i: (i, 0)),
            pl.BlockSpec((_BLK, 1), lambda i: (i, 0)),
        ],
        out_shape=[
            jax.ShapeDtypeStruct((N, D), jnp.float32),
            jax.ShapeDtypeStruct((N, 1), jnp.float32),
        ],
    )(hw, degp, degp)


def _tc_b_body(s0_ref, s1_ref, g1_ref, dinv_ref, b1_ref, w2_ref, g2_ref):
    dinv = dinv_ref[...]
    h = dinv * (s0_ref[...] + s1_ref[...] + g1_ref[...]) + b1_ref[...]
    h = jnp.maximum(h, 0.0)
    g2_ref[...] = jnp.dot(h, w2_ref[...],
                          preferred_element_type=jnp.float32) * dinv


def _tc_b(s1, g1, dinv, b1r, W2):
    return pl.pallas_call(
        _tc_b_body,
        grid=(_GRID,),
        in_specs=[
            pl.BlockSpec((_BLK, D), lambda i: (i, 0)),
            pl.BlockSpec((_BLK, D), lambda i: (i + _GRID, 0)),
            pl.BlockSpec((_BLK, D), lambda i: (i, 0)),
            pl.BlockSpec((_BLK, 1), lambda i: (i, 0)),
            pl.BlockSpec((1, D), lambda i: (0, 0)),
            pl.BlockSpec((D, D), lambda i: (0, 0)),
        ],
        out_specs=pl.BlockSpec((_BLK, D), lambda i: (i, 0)),
        out_shape=jax.ShapeDtypeStruct((N, D), jnp.float32),
    )(s1, s1, g1, dinv, b1r, W2)


def _tc_c_body(s0_ref, s1_ref, g2_ref, dinv_ref, b2_ref, out_ref):
    out_ref[...] = (dinv_ref[...] * (s0_ref[...] + s1_ref[...] + g2_ref[...])
                    + b2_ref[...])


def _tc_c(s2, g2, dinv, b2r):
    return pl.pallas_call(
        _tc_c_body,
        grid=(_GRID,),
        in_specs=[
            pl.BlockSpec((_BLK, D), lambda i: (i, 0)),
            pl.BlockSpec((_BLK, D), lambda i: (i + _GRID, 0)),
            pl.BlockSpec((_BLK, D), lambda i: (i, 0)),
            pl.BlockSpec((_BLK, 1), lambda i: (i, 0)),
            pl.BlockSpec((1, D), lambda i: (0, 0)),
        ],
        out_specs=pl.BlockSpec((_BLK, D), lambda i: (i, 0)),
        out_shape=jax.ShapeDtypeStruct((N, D), jnp.float32),
    )(s2, s2, g2, dinv, b2r)


def kernel(x, edge_index, W1, b1, W2, b2):
    pad_i = jnp.arange(EPAD, dtype=jnp.int32)
    src = jnp.concatenate(
        [edge_index[0].astype(jnp.int32), N + pad_i % PADROWS]
    ).reshape(NCHUNKS_P, CHUNK)
    dst = jnp.concatenate(
        [edge_index[1].astype(jnp.int32), pad_i % N]
    ).reshape(NCHUNKS_P, CHUNK)
    zeros_vec = jnp.zeros((N,), jnp.float32)
    zeros_tab = jnp.zeros((N, D), jnp.float32)
    tab_pad = jnp.zeros((PADROWS, D), jnp.float32)
    b1r = b1.reshape(1, D)
    b2r = b2.reshape(1, D)

    deg_kernel = _make_deg_kernel()
    gs_kernel = _make_gs_kernel()
    hw1 = _tc_mm(x, W1)
    degp = deg_kernel(dst, zeros_vec).reshape(2 * N, 1)
    g1, dinv = _tc_a(hw1, degp)
    s1 = gs_kernel(src, dst, jnp.concatenate([g1, tab_pad]), zeros_tab)
    g2 = _tc_b(s1, g1, dinv, b1r, W2)
    s2 = gs_kernel(src, dst, jnp.concatenate([g2, tab_pad]), zeros_tab)
    return _tc_c(s2, g2, dinv, b2r)


# speedup vs baseline: 29.0180x; 1.0419x over previous
import functools

import jax
import jax.numpy as jnp
from jax import lax
from jax.experimental import pallas as pl
from jax.experimental.pallas import tpu as pltpu
from jax.experimental.pallas import tpu_sc as plsc

N = 10000
E = 320000
D = 128
CHUNK = 128
NC = 2
NS = 16
NW = NC * NS
EPB = 80
NCHUNKS_P = NW * EPB
EPAD = NCHUNKS_P * CHUNK - E
NREAL_CHUNKS = E // CHUNK
PADROWS = 64
NACC = N + PADROWS
QCH = 40


@functools.cache
def _make_deg_kernel():
    mesh = plsc.VectorSubcoreMesh(core_axis_name="c", subcore_axis_name="s")
    return functools.partial(
        pl.kernel,
        mesh=mesh,
        out_type=jax.ShapeDtypeStruct((2 * N,), jnp.float32),
        scratch_types=[
            pltpu.VMEM((EPB, CHUNK), jnp.int32),
            pltpu.VMEM((CHUNK,), jnp.float32),
            pltpu.VMEM((N,), jnp.float32),
            pltpu.VMEM_SHARED((N,), jnp.float32),
        ],
    )(_deg_body)


def _deg_body(dst_hbm, out_hbm, idx_d, ones_v, bounce_v, acc):
    c = lax.axis_index("c")
    s = lax.axis_index("s")
    wid = s * NC + c

    for k in range(CHUNK // 16):
        ones_v[pl.ds(k * 16, 16)] = jnp.ones((16,), jnp.float32)

    pltpu.sync_copy(dst_hbm.at[pl.ds(wid * EPB, EPB)], idx_d)

    @pl.when(s == 0)
    def _():
        def zf(k, carry):
            bounce_v[pl.ds(k * 16, 16)] = jnp.zeros((16,), jnp.float32)
            return carry

        lax.fori_loop(0, N // 16, zf, 0)
        pltpu.sync_copy(bounce_v, acc)

    plsc.subcore_barrier()

    nreal = jnp.clip(NREAL_CHUNKS - wid * EPB, 0, EPB)

    def fire(j, carry):
        pltpu.sync_copy(ones_v, acc.at[idx_d.at[j]], add=True)
        return carry

    lax.fori_loop(0, nreal, fire, 0)

    plsc.subcore_barrier()

    @pl.when(s == 0)
    def _():
        pltpu.sync_copy(acc, bounce_v)
        pltpu.sync_copy(bounce_v, out_hbm.at[pl.ds(c * N, N)])


@functools.cache
def _make_gs_kernel():
    mesh = plsc.VectorSubcoreMesh(core_axis_name="c", subcore_axis_name="s")
    return functools.partial(
        pl.kernel,
        mesh=mesh,
        out_type=jax.ShapeDtypeStruct((2 * N, D), jnp.float32),
        scratch_types=[
            pltpu.VMEM((QCH, CHUNK), jnp.int32),
            pltpu.VMEM((QCH, CHUNK), jnp.int32),
            pltpu.VMEM((CHUNK, D), jnp.float32),
            pltpu.VMEM((CHUNK, D), jnp.float32),
            pltpu.VMEM_SHARED((NACC, D), jnp.float32),
            pltpu.SemaphoreType.DMA,
        ],
    )(_gs_body)


def _gs_body(src_hbm, dst_hbm, table_hbm, out_hbm,
             idx_s, idx_d, rows0, rows1, acc, gsem):
    c = lax.axis_index("c")
    s = lax.axis_index("s")
    wid = s * NC + c

    def zrow(r, carry):
        for k in range(D // 16):
            rows0[r, pl.ds(k * 16, 16)] = jnp.zeros((16,), jnp.float32)
        return carry

    lax.fori_loop(0, CHUNK, zrow, 0)

    @pl.when(s < NS - 1)
    def _():
        for k in range(5):
            pltpu.sync_copy(rows0, acc.at[pl.ds(s * 640 + k * 128, 128)])

    @pl.when(s == NS - 1)
    def _():
        for k in range(3):
            pltpu.sync_copy(rows0, acc.at[pl.ds(9600 + k * 128, 128)])
        pltpu.sync_copy(rows0.at[pl.ds(0, 80)],
                        acc.at[pl.ds(9984, 80)])

    plsc.subcore_barrier()

    for q in range(EPB // QCH):
        qbase = wid * EPB + q * QCH
        pltpu.sync_copy(src_hbm.at[pl.ds(qbase, QCH)], idx_s)
        pltpu.sync_copy(dst_hbm.at[pl.ds(qbase, QCH)], idx_d)

        pltpu.async_copy(table_hbm.at[idx_s.at[0]], rows0, gsem)

        def outer(i, carry):
            j = 2 * i
            pltpu.make_async_copy(table_hbm.at[idx_s.at[j]], rows0, gsem).wait()
            pltpu.async_copy(table_hbm.at[idx_s.at[j + 1]], rows1, gsem)
            pltpu.sync_copy(rows0, acc.at[idx_d.at[j]], add=True)

            pltpu.make_async_copy(table_hbm.at[idx_s.at[j + 1]], rows1,
                                  gsem).wait()

            @pl.when(j + 2 < QCH)
            def _():
                pltpu.async_copy(table_hbm.at[idx_s.at[j + 2]], rows0, gsem)

            pltpu.sync_copy(rows1, acc.at[idx_d.at[j + 1]], add=True)
            return carry

        lax.fori_loop(0, QCH // 2, outer, 0)

    plsc.subcore_barrier()

    @pl.when(s < NS - 1)
    def _():
        base = s * 640
        pltpu.sync_copy(acc.at[pl.ds(base, 640)],
                        out_hbm.at[pl.ds(c * N + base, 640)])

    @pl.when(s == NS - 1)
    def _():
        pltpu.sync_copy(acc.at[pl.ds(9600, 400)],
                        out_hbm.at[pl.ds(c * N + 9600, 400)])


_BLK = 1000
_GRID = N // _BLK


def _tc_mm_body(x_ref, w_ref, o_ref):
    o_ref[...] = jnp.dot(x_ref[...], w_ref[...],
                         preferred_element_type=jnp.float32)


def _tc_mm(x, W1):
    return pl.pallas_call(
        _tc_mm_body,
        grid=(_GRID,),
        in_specs=[
            pl.BlockSpec((_BLK, D), lambda i: (i, 0)),
            pl.BlockSpec((D, D), lambda i: (0, 0)),
        ],
        out_specs=pl.BlockSpec((_BLK, D), lambda i: (i, 0)),
        out_shape=jax.ShapeDtypeStruct((N, D), jnp.float32),
    )(x, W1)


def _tc_a_body(hw_ref, d0_ref, d1_ref, g_ref, dinv_ref):
    deg = d0_ref[...] + d1_ref[...] + 1.0
    dinv = lax.rsqrt(deg)
    g_ref[...] = hw_ref[...] * dinv
    dinv_ref[...] = dinv


def _tc_a(hw, degp):
    return pl.pallas_call(
        _tc_a_body,
        grid=(_GRID,),
        in_specs=[
            pl.BlockSpec((_BLK, D), lambda i: (i, 0)),
            pl.BlockSpec((_BLK, 1), lambda i: (i, 0)),
            pl.BlockSpec((_BLK, 1), lambda i: (i + _GRID, 0)),
        ],
        out_specs=[
            pl.BlockSpec((_BLK, D), lambda i: (i, 0)),
            pl.BlockSpec((_BLK, 1), lambda i: (i, 0)),
        ],
        out_shape=[
            jax.ShapeDtypeStruct((N, D), jnp.float32),
            jax.ShapeDtypeStruct((N, 1), jnp.float32),
        ],
    )(hw, degp, degp)


def _tc_b_body(s0_ref, s1_ref, g1_ref, dinv_ref, b1_ref, w2_ref, g2_ref):
    dinv = dinv_ref[...]
    h = dinv * (s0_ref[...] + s1_ref[...] + g1_ref[...]) + b1_ref[...]
    h = jnp.maximum(h, 0.0)
    g2_ref[...] = jnp.dot(h, w2_ref[...],
                          preferred_element_type=jnp.float32) * dinv


def _tc_b(s1, g1, dinv, b1r, W2):
    return pl.pallas_call(
        _tc_b_body,
        grid=(_GRID,),
        in_specs=[
            pl.BlockSpec((_BLK, D), lambda i: (i, 0)),
            pl.BlockSpec((_BLK, D), lambda i: (i + _GRID, 0)),
            pl.BlockSpec((_BLK, D), lambda i: (i, 0)),
            pl.BlockSpec((_BLK, 1), lambda i: (i, 0)),
            pl.BlockSpec((1, D), lambda i: (0, 0)),
            pl.BlockSpec((D, D), lambda i: (0, 0)),
        ],
        out_specs=pl.BlockSpec((_BLK, D), lambda i: (i, 0)),
        out_shape=jax.ShapeDtypeStruct((N, D), jnp.float32),
    )(s1, s1, g1, dinv, b1r, W2)


def _tc_c_body(s0_ref, s1_ref, g2_ref, dinv_ref, b2_ref, out_ref):
    out_ref[...] = (dinv_ref[...] * (s0_ref[...] + s1_ref[...] + g2_ref[...])
                    + b2_ref[...])


def _tc_c(s2, g2, dinv, b2r):
    return pl.pallas_call(
        _tc_c_body,
        grid=(_GRID,),
        in_specs=[
            pl.BlockSpec((_BLK, D), lambda i: (i, 0)),
            pl.BlockSpec((_BLK, D), lambda i: (i + _GRID, 0)),
            pl.BlockSpec((_BLK, D), lambda i: (i, 0)),
            pl.BlockSpec((_BLK, 1), lambda i: (i, 0)),
            pl.BlockSpec((1, D), lambda i: (0, 0)),
        ],
        out_specs=pl.BlockSpec((_BLK, D), lambda i: (i, 0)),
        out_shape=jax.ShapeDtypeStruct((N, D), jnp.float32),
    )(s2, s2, g2, dinv, b2r)


def kernel(x, edge_index, W1, b1, W2, b2):
    pad_i = jnp.arange(EPAD, dtype=jnp.int32)
    src = jnp.concatenate(
        [edge_index[0].astype(jnp.int32), pad_i % N]
    ).reshape(NCHUNKS_P, CHUNK)
    dst = jnp.concatenate(
        [edge_index[1].astype(jnp.int32), N + pad_i % PADROWS]
    ).reshape(NCHUNKS_P, CHUNK)
    b1r = b1.reshape(1, D)
    b2r = b2.reshape(1, D)

    deg_kernel = _make_deg_kernel()
    gs_kernel = _make_gs_kernel()
    hw1 = _tc_mm(x, W1)
    degp = deg_kernel(dst).reshape(2 * N, 1)
    g1, dinv = _tc_a(hw1, degp)
    s1 = gs_kernel(src, dst, g1)
    g2 = _tc_b(s1, g1, dinv, b1r, W2)
    s2 = gs_kernel(src, dst, g2)
    return _tc_c(s2, g2, dinv, b2r)


# speedup vs baseline: 29.6038x; 1.0202x over previous
import functools

import jax
import jax.numpy as jnp
from jax import lax
from jax.experimental import pallas as pl
from jax.experimental.pallas import tpu as pltpu
from jax.experimental.pallas import tpu_sc as plsc

N = 10000
E = 320000
D = 128
CHUNK = 128
NC = 2
NS = 16
NW = NC * NS
EPB = 80
NCHUNKS_P = NW * EPB
EPAD = NCHUNKS_P * CHUNK - E
NREAL_CHUNKS = E // CHUNK
PADROWS = 64
NACC = N + PADROWS
QCH = 40


@functools.cache
def _make_deg_kernel():
    mesh = plsc.VectorSubcoreMesh(core_axis_name="c", subcore_axis_name="s")
    return functools.partial(
        pl.kernel,
        mesh=mesh,
        out_type=jax.ShapeDtypeStruct((2 * N,), jnp.float32),
        scratch_types=[
            pltpu.VMEM((EPB, CHUNK), jnp.int32),
            pltpu.VMEM((CHUNK,), jnp.float32),
            pltpu.VMEM((N,), jnp.float32),
            pltpu.VMEM_SHARED((N,), jnp.float32),
        ],
    )(_deg_body)


def _deg_body(dst_hbm, out_hbm, idx_d, ones_v, bounce_v, acc):
    c = lax.axis_index("c")
    s = lax.axis_index("s")
    wid = s * NC + c

    for k in range(CHUNK // 16):
        ones_v[pl.ds(k * 16, 16)] = jnp.ones((16,), jnp.float32)

    pltpu.sync_copy(dst_hbm.at[pl.ds(wid * EPB, EPB)], idx_d)

    @pl.when(s == 0)
    def _():
        def zf(k, carry):
            for u in range(8):
                bounce_v[pl.ds(k * 128 + u * 16, 16)] = jnp.zeros(
                    (16,), jnp.float32)
            return carry

        lax.fori_loop(0, N // 128, zf, 0)
        bounce_v[pl.ds(N - N % 128, 16)] = jnp.zeros((16,), jnp.float32)
        pltpu.sync_copy(bounce_v, acc)

    plsc.subcore_barrier()

    nreal = jnp.clip(NREAL_CHUNKS - wid * EPB, 0, EPB)

    def fire(j, carry):
        pltpu.sync_copy(ones_v, acc.at[idx_d.at[j]], add=True)
        return carry

    lax.fori_loop(0, nreal, fire, 0)

    plsc.subcore_barrier()

    @pl.when(s == 0)
    def _():
        pltpu.sync_copy(acc, bounce_v)
        pltpu.sync_copy(bounce_v, out_hbm.at[pl.ds(c * N, N)])


@functools.cache
def _make_gs_kernel():
    mesh = plsc.VectorSubcoreMesh(core_axis_name="c", subcore_axis_name="s")
    return functools.partial(
        pl.kernel,
        mesh=mesh,
        out_type=jax.ShapeDtypeStruct((2 * N, D), jnp.float32),
        scratch_types=[
            pltpu.VMEM((CHUNK,), jnp.int32),
            pltpu.VMEM((CHUNK,), jnp.int32),
            pltpu.VMEM((EPB, CHUNK), jnp.int32),
            pltpu.VMEM((CHUNK, D), jnp.float32),
            pltpu.VMEM((CHUNK, D), jnp.float32),
            pltpu.VMEM_SHARED((NACC, D), jnp.float32),
            pltpu.SemaphoreType.DMA,
            pltpu.SemaphoreType.DMA,
            pltpu.SemaphoreType.DMA,
        ],
    )(_gs_body)


def _gs_body(src_hbm, dst_hbm, table_hbm, out_hbm,
             sidx0, sidx1, idx_d, rows0, rows1, acc, gsem, is0, is1):
    c = lax.axis_index("c")
    s = lax.axis_index("s")
    wid = s * NC + c

    def zrow(r, carry):
        for k in range(D // 16):
            rows0[r, pl.ds(k * 16, 16)] = jnp.zeros((16,), jnp.float32)
        return carry

    lax.fori_loop(0, CHUNK, zrow, 0)

    @pl.when(s < NS - 1)
    def _():
        for k in range(5):
            pltpu.sync_copy(rows0, acc.at[pl.ds(s * 640 + k * 128, 128)])

    @pl.when(s == NS - 1)
    def _():
        for k in range(3):
            pltpu.sync_copy(rows0, acc.at[pl.ds(9600 + k * 128, 128)])
        pltpu.sync_copy(rows0.at[pl.ds(0, 80)],
                        acc.at[pl.ds(9984, 80)])

    plsc.subcore_barrier()

    base = wid * EPB
    sidx = (sidx0, sidx1)
    isem = (is0, is1)
    pltpu.sync_copy(dst_hbm.at[pl.ds(base, EPB)], idx_d)
    pltpu.sync_copy(src_hbm.at[base], sidx0)
    pltpu.async_copy(src_hbm.at[base + 1], sidx1, is1)
    pltpu.async_copy(table_hbm.at[sidx0], rows0, gsem)

    def outer(i, carry):
        for b in range(2):
            j = 2 * i + b
            rows_b = rows0 if b == 0 else rows1
            rows_n = rows1 if b == 0 else rows0
            pltpu.make_async_copy(table_hbm.at[sidx[b]], rows_b, gsem).wait()

            @pl.when(j + 2 < EPB)
            def _():
                pltpu.async_copy(src_hbm.at[base + j + 2], sidx[b], isem[b])

            @pl.when(j + 1 < EPB)
            def _():
                pltpu.make_async_copy(src_hbm.at[base + j + 1],
                                      sidx[1 - b], isem[1 - b]).wait()
                pltpu.async_copy(table_hbm.at[sidx[1 - b]], rows_n, gsem)

            pltpu.sync_copy(rows_b, acc.at[idx_d.at[j]], add=True)
        return carry

    lax.fori_loop(0, EPB // 2, outer, 0)

    plsc.subcore_barrier()

    @pl.when(s < NS - 1)
    def _():
        base = s * 640
        pltpu.sync_copy(acc.at[pl.ds(base, 640)],
                        out_hbm.at[pl.ds(c * N + base, 640)])

    @pl.when(s == NS - 1)
    def _():
        pltpu.sync_copy(acc.at[pl.ds(9600, 400)],
                        out_hbm.at[pl.ds(c * N + 9600, 400)])


_BLK = 1000
_GRID = N // _BLK


def _tc_mm_body(x_ref, w_ref, o_ref):
    o_ref[...] = jnp.dot(x_ref[...], w_ref[...],
                         preferred_element_type=jnp.float32)


def _tc_mm(x, W1):
    return pl.pallas_call(
        _tc_mm_body,
        grid=(_GRID,),
        in_specs=[
            pl.BlockSpec((_BLK, D), lambda i: (i, 0)),
            pl.BlockSpec((D, D), lambda i: (0, 0)),
        ],
        out_specs=pl.BlockSpec((_BLK, D), lambda i: (i, 0)),
        out_shape=jax.ShapeDtypeStruct((N, D), jnp.float32),
    )(x, W1)


def _tc_a_body(hw_ref, d0_ref, d1_ref, g_ref, dinv_ref):
    deg = d0_ref[...] + d1_ref[...] + 1.0
    dinv = lax.rsqrt(deg)
    g_ref[...] = hw_ref[...] * dinv
    dinv_ref[...] = dinv


def _tc_a(hw, degp):
    return pl.pallas_call(
        _tc_a_body,
        grid=(_GRID,),
        in_specs=[
            pl.BlockSpec((_BLK, D), lambda i: (i, 0)),
            pl.BlockSpec((_BLK, 1), lambda i: (i, 0)),
            pl.BlockSpec((_BLK, 1), lambda i: (i + _GRID, 0)),
        ],
        out_specs=[
            pl.BlockSpec((_BLK, D), lambda i: (i, 0)),
            pl.BlockSpec((_BLK, 1), lambda i: (i, 0)),
        ],
        out_shape=[
            jax.ShapeDtypeStruct((N, D), jnp.float32),
            jax.ShapeDtypeStruct((N, 1), jnp.float32),
        ],
    )(hw, degp, degp)


def _tc_b_body(s0_ref, s1_ref, g1_ref, dinv_ref, b1_ref, w2_ref, g2_ref):
    dinv = dinv_ref[...]
    h = dinv * (s0_ref[...] + s1_ref[...] + g1_ref[...]) + b1_ref[...]
    h = jnp.maximum(h, 0.0)
    g2_ref[...] = jnp.dot(h, w2_ref[...],
                          preferred_element_type=jnp.float32) * dinv


def _tc_b(s1, g1, dinv, b1r, W2):
    return pl.pallas_call(
        _tc_b_body,
        grid=(_GRID,),
        in_specs=[
            pl.BlockSpec((_BLK, D), lambda i: (i, 0)),
            pl.BlockSpec((_BLK, D), lambda i: (i + _GRID, 0)),
            pl.BlockSpec((_BLK, D), lambda i: (i, 0)),
            pl.BlockSpec((_BLK, 1), lambda i: (i, 0)),
            pl.BlockSpec((1, D), lambda i: (0, 0)),
            pl.BlockSpec((D, D), lambda i: (0, 0)),
        ],
        out_specs=pl.BlockSpec((_BLK, D), lambda i: (i, 0)),
        out_shape=jax.ShapeDtypeStruct((N, D), jnp.float32),
    )(s1, s1, g1, dinv, b1r, W2)


def _tc_c_body(s0_ref, s1_ref, g2_ref, dinv_ref, b2_ref, out_ref):
    out_ref[...] = (dinv_ref[...] * (s0_ref[...] + s1_ref[...] + g2_ref[...])
                    + b2_ref[...])


def _tc_c(s2, g2, dinv, b2r):
    return pl.pallas_call(
        _tc_c_body,
        grid=(_GRID,),
        in_specs=[
            pl.BlockSpec((_BLK, D), lambda i: (i, 0)),
            pl.BlockSpec((_BLK, D), lambda i: (i + _GRID, 0)),
            pl.BlockSpec((_BLK, D), lambda i: (i, 0)),
            pl.BlockSpec((_BLK, 1), lambda i: (i, 0)),
            pl.BlockSpec((1, D), lambda i: (0, 0)),
        ],
        out_specs=pl.BlockSpec((_BLK, D), lambda i: (i, 0)),
        out_shape=jax.ShapeDtypeStruct((N, D), jnp.float32),
    )(s2, s2, g2, dinv, b2r)


def kernel(x, edge_index, W1, b1, W2, b2):
    pad_i = jnp.arange(EPAD, dtype=jnp.int32)
    src = jnp.concatenate(
        [edge_index[0].astype(jnp.int32), pad_i % N]
    ).reshape(NCHUNKS_P, CHUNK)
    dst = jnp.concatenate(
        [edge_index[1].astype(jnp.int32), N + pad_i % PADROWS]
    ).reshape(NCHUNKS_P, CHUNK)
    b1r = b1.reshape(1, D)
    b2r = b2.reshape(1, D)

    deg_kernel = _make_deg_kernel()
    gs_kernel = _make_gs_kernel()
    hw1 = _tc_mm(x, W1)
    degp = deg_kernel(dst).reshape(2 * N, 1)
    g1, dinv = _tc_a(hw1, degp)
    s1 = gs_kernel(src, dst, g1)
    g2 = _tc_b(s1, g1, dinv, b1r, W2)
    s2 = gs_kernel(src, dst, g2)
    return _tc_c(s2, g2, dinv, b2r)
